# baseline (speedup 1.0000x reference)
import numpy as np
import jax
import jax.numpy as jnp
from jax.experimental import pallas as pl

B, N, S, MAXLEN, D, H = 8, 2048, 4, 16, 128, 128
ROWS = B * N
BLK = 256

_SEG = np.zeros((MAXLEN * D, MAXLEN), dtype=np.float32)
for _m in range(MAXLEN):
    _SEG[_m * D:(_m + 1) * D, _m] = 1.0
_SEG = jnp.asarray(_SEG)


def _fused_kernel(src_ref, neigh_ref, w_ref, seg_ref, out_ref):
    neigh = neigh_ref[...]
    src = src_ref[...]

    absn = jnp.abs(neigh)
    row_abs = jnp.dot(absn, seg_ref[...], preferred_element_type=jnp.float32)
    count = jnp.sum((row_abs > 0.0).astype(jnp.float32), axis=-1)
    denom = jnp.maximum(count, 1.0)

    nsum = neigh[:, 0:D]
    for m in range(1, MAXLEN):
        nsum = nsum + neigh[:, m * D:(m + 1) * D]
    ssum = src[:, 0:D]
    for s in range(1, S):
        ssum = ssum + src[:, s * D:(s + 1) * D]

    hidden = ssum + nsum / denom[:, None]
    out = jnp.dot(hidden, w_ref[...], preferred_element_type=jnp.float32)
    out_ref[...] = jnp.where(out >= 0.0, out, 0.01 * out)


@jax.jit
def _run(src, neigh, weight):
    src = src.reshape(ROWS, S * D)
    neigh = neigh.reshape(ROWS, MAXLEN * D)
    grid = (ROWS // BLK,)
    return pl.pallas_call(
        _fused_kernel,
        grid=grid,
        in_specs=[
            pl.BlockSpec((BLK, S * D), lambda i: (i, 0)),
            pl.BlockSpec((BLK, MAXLEN * D), lambda i: (i, 0)),
            pl.BlockSpec((D, H), lambda i: (0, 0)),
            pl.BlockSpec((MAXLEN * D, MAXLEN), lambda i: (0, 0)),
        ],
        out_specs=pl.BlockSpec((BLK, H), lambda i: (i, 0)),
        out_shape=jax.ShapeDtypeStruct((ROWS, H), jnp.float32),
    )(src, neigh, weight, _SEG)


def kernel(src_node_features, neigh_node_features, src_nodes, weight):
    return _run(src_node_features, neigh_node_features, weight)


# speedup vs baseline: 1.0020x; 1.0020x over previous
import numpy as np
import jax
import jax.numpy as jnp
from jax.experimental import pallas as pl

B, N, S, MAXLEN, D, H = 8, 2048, 4, 16, 128, 128
ROWS = B * N
BLK = 256

_SEG_NP = np.zeros((MAXLEN * D, MAXLEN), dtype=np.float32)
for _m in range(MAXLEN):
    _SEG_NP[_m * D:(_m + 1) * D, _m] = 1.0


def _fused_kernel(src_ref, neigh_ref, w_ref, seg_ref, out_ref):
    neigh = neigh_ref[...]
    src = src_ref[...]

    absn = jnp.abs(neigh)
    row_abs = jnp.dot(absn, seg_ref[...], preferred_element_type=jnp.float32)
    count = jnp.sum((row_abs > 0.0).astype(jnp.float32), axis=-1)
    denom = jnp.maximum(count, 1.0)

    nsum = neigh[:, 0:D]
    for m in range(1, MAXLEN):
        nsum = nsum + neigh[:, m * D:(m + 1) * D]
    ssum = src[:, 0:D]
    for s in range(1, S):
        ssum = ssum + src[:, s * D:(s + 1) * D]

    hidden = ssum + nsum / denom[:, None]
    out = jnp.dot(hidden, w_ref[...], preferred_element_type=jnp.float32)
    out_ref[...] = jnp.where(out >= 0.0, out, 0.01 * out)


@jax.jit
def _run(src, neigh, weight):
    src = src.reshape(ROWS, S * D)
    neigh = neigh.reshape(ROWS, MAXLEN * D)
    seg = jnp.asarray(_SEG_NP)
    grid = (ROWS // BLK,)
    return pl.pallas_call(
        _fused_kernel,
        grid=grid,
        in_specs=[
            pl.BlockSpec((BLK, S * D), lambda i: (i, 0)),
            pl.BlockSpec((BLK, MAXLEN * D), lambda i: (i, 0)),
            pl.BlockSpec((D, H), lambda i: (0, 0)),
            pl.BlockSpec((MAXLEN * D, MAXLEN), lambda i: (0, 0)),
        ],
        out_specs=pl.BlockSpec((BLK, H), lambda i: (i, 0)),
        out_shape=jax.ShapeDtypeStruct((ROWS, H), jnp.float32),
    )(src, neigh, weight, seg)


def kernel(src_node_features, neigh_node_features, src_nodes, weight):
    return _run(src_node_features, neigh_node_features, weight)


# speedup vs baseline: 2.2798x; 2.2752x over previous
import jax
import jax.numpy as jnp
from jax.experimental import pallas as pl

B, N, S, MAXLEN, D, H = 8, 2048, 4, 16, 128, 128
ROWS = B * N
BLK = 256
TR = 32


def _fused_kernel(src_ref, neigh_ref, w_ref, out_ref):
    neigh = neigh_ref[...]
    src = src_ref[...]

    sabs = jnp.sum(jnp.abs(neigh), axis=-1)
    count = jnp.sum((sabs > 0.0).astype(jnp.float32), axis=-1)
    denom = jnp.maximum(count, 1.0)

    half = neigh[:, 0:8, :] + neigh[:, 8:16, :]
    nsum = jnp.sum(half, axis=1)
    ssum = jnp.sum(src, axis=1)

    hidden = ssum + nsum / denom[:, None]
    out = jnp.dot(hidden, w_ref[...], preferred_element_type=jnp.float32)
    out_ref[...] = jnp.where(out >= 0.0, out, 0.01 * out)


@jax.jit
def _run(src, neigh, weight):
    src = src.reshape(ROWS, S, D)
    neigh = neigh.reshape(ROWS, MAXLEN, D)
    grid = (ROWS // BLK,)
    return pl.pallas_call(
        _fused_kernel,
        grid=grid,
        in_specs=[
            pl.BlockSpec((BLK, S, D), lambda i: (i, 0, 0)),
            pl.BlockSpec((BLK, MAXLEN, D), lambda i: (i, 0, 0)),
            pl.BlockSpec((D, H), lambda i: (0, 0)),
        ],
        out_specs=pl.BlockSpec((BLK, H), lambda i: (i, 0)),
        out_shape=jax.ShapeDtypeStruct((ROWS, H), jnp.float32),
    )(src, neigh, weight)


def kernel(src_node_features, neigh_node_features, src_nodes, weight):
    return _run(src_node_features, neigh_node_features, weight)


# speedup vs baseline: 2.9496x; 1.2938x over previous
import jax
import jax.numpy as jnp
from jax.experimental import pallas as pl

B, N, S, MAXLEN, D, H = 8, 2048, 4, 16, 128, 128
ROWS = B * N
BLK = 512
TR = 32


def _fused_kernel(src_ref, neigh_ref, w_ref, out_ref):
    neigh = neigh_ref[...]
    src = src_ref[...]

    sabs = jnp.sum(jnp.abs(neigh), axis=-1)
    ind = (sabs > 0.0).astype(jnp.float32)
    cnt = jnp.dot(ind, jnp.ones((MAXLEN, D), jnp.float32),
                  preferred_element_type=jnp.float32)
    rdenom = 1.0 / jnp.maximum(cnt, 1.0)

    half = neigh[:, 0:8, :] + neigh[:, 8:16, :]
    nsum = jnp.sum(half, axis=1)
    ssum = ((src[:, 0, :] + src[:, 1, :])
            + (src[:, 2, :] + src[:, 3, :]))

    hidden = ssum + nsum * rdenom
    out = jnp.dot(hidden, w_ref[...], preferred_element_type=jnp.float32)
    out_ref[...] = jnp.where(out >= 0.0, out, 0.01 * out)


@jax.jit
def _run(src, neigh, weight):
    src = src.reshape(ROWS, S, D)
    neigh = neigh.reshape(ROWS, MAXLEN, D)
    grid = (ROWS // BLK,)
    return pl.pallas_call(
        _fused_kernel,
        grid=grid,
        in_specs=[
            pl.BlockSpec((BLK, S, D), lambda i: (i, 0, 0)),
            pl.BlockSpec((BLK, MAXLEN, D), lambda i: (i, 0, 0)),
            pl.BlockSpec((D, H), lambda i: (0, 0)),
        ],
        out_specs=pl.BlockSpec((BLK, H), lambda i: (i, 0)),
        out_shape=jax.ShapeDtypeStruct((ROWS, H), jnp.float32),
    )(src, neigh, weight)


def kernel(src_node_features, neigh_node_features, src_nodes, weight):
    return _run(src_node_features, neigh_node_features, weight)


# speedup vs baseline: 3.2835x; 1.1132x over previous
import jax
import jax.numpy as jnp
from jax.experimental import pallas as pl

B, N, S, MAXLEN, D, H = 8, 2048, 4, 16, 128, 128
ROWS = B * N
BLK = 1024
TR = 32


def _fused_kernel(src_ref, neigh_ref, w_ref, out_ref):
    neigh = neigh_ref[...]
    src = src_ref[...]

    sabs = jnp.sum(jnp.abs(neigh), axis=-1)
    ind = (sabs > 0.0).astype(jnp.float32)
    cnt = jnp.dot(ind, jnp.ones((MAXLEN, D), jnp.float32),
                  preferred_element_type=jnp.float32)
    rdenom = 1.0 / jnp.maximum(cnt, 1.0)

    half = neigh[:, 0:8, :] + neigh[:, 8:16, :]
    nsum = jnp.sum(half, axis=1)
    ssum = ((src[:, 0, :] + src[:, 1, :])
            + (src[:, 2, :] + src[:, 3, :]))

    hidden = ssum + nsum * rdenom
    out = jnp.dot(hidden, w_ref[...], preferred_element_type=jnp.float32)
    out_ref[...] = jnp.where(out >= 0.0, out, 0.01 * out)


@jax.jit
def _run(src, neigh, weight):
    src = src.reshape(ROWS, S, D)
    neigh = neigh.reshape(ROWS, MAXLEN, D)
    grid = (ROWS // BLK,)
    return pl.pallas_call(
        _fused_kernel,
        grid=grid,
        in_specs=[
            pl.BlockSpec((BLK, S, D), lambda i: (i, 0, 0)),
            pl.BlockSpec((BLK, MAXLEN, D), lambda i: (i, 0, 0)),
            pl.BlockSpec((D, H), lambda i: (0, 0)),
        ],
        out_specs=pl.BlockSpec((BLK, H), lambda i: (i, 0)),
        out_shape=jax.ShapeDtypeStruct((ROWS, H), jnp.float32),
    )(src, neigh, weight)


def kernel(src_node_features, neigh_node_features, src_nodes, weight):
    return _run(src_node_features, neigh_node_features, weight)


# speedup vs baseline: 3.3774x; 1.0286x over previous
import jax
import jax.numpy as jnp
from jax.experimental import pallas as pl

B, N, S, MAXLEN, D, H = 8, 2048, 4, 16, 128, 128
ROWS = B * N
BLK = 2048
TR = 32


def _fused_kernel(src_ref, neigh_ref, w_ref, out_ref):
    neigh = neigh_ref[...]
    src = src_ref[...]

    sabs = jnp.sum(jnp.abs(neigh), axis=-1)
    ind = (sabs > 0.0).astype(jnp.float32)
    cnt = jnp.dot(ind, jnp.ones((MAXLEN, D), jnp.float32),
                  preferred_element_type=jnp.float32)
    rdenom = 1.0 / jnp.maximum(cnt, 1.0)

    half = neigh[:, 0:8, :] + neigh[:, 8:16, :]
    nsum = jnp.sum(half, axis=1)
    ssum = ((src[:, 0, :] + src[:, 1, :])
            + (src[:, 2, :] + src[:, 3, :]))

    hidden = ssum + nsum * rdenom
    out = jnp.dot(hidden, w_ref[...], preferred_element_type=jnp.float32)
    out_ref[...] = jnp.where(out >= 0.0, out, 0.01 * out)


@jax.jit
def _run(src, neigh, weight):
    src = src.reshape(ROWS, S, D)
    neigh = neigh.reshape(ROWS, MAXLEN, D)
    grid = (ROWS // BLK,)
    return pl.pallas_call(
        _fused_kernel,
        grid=grid,
        in_specs=[
            pl.BlockSpec((BLK, S, D), lambda i: (i, 0, 0)),
            pl.BlockSpec((BLK, MAXLEN, D), lambda i: (i, 0, 0)),
            pl.BlockSpec((D, H), lambda i: (0, 0)),
        ],
        out_specs=pl.BlockSpec((BLK, H), lambda i: (i, 0)),
        out_shape=jax.ShapeDtypeStruct((ROWS, H), jnp.float32),
    )(src, neigh, weight)


def kernel(src_node_features, neigh_node_features, src_nodes, weight):
    return _run(src_node_features, neigh_node_features, weight)
